# Initial kernel scaffold; baseline (speedup 1.0000x reference)
import functools

import jax
import jax.numpy as jnp
from jax import lax
from jax.experimental import pallas as pl
from jax.experimental.pallas import tpu as pltpu
from jax.experimental.pallas import tpu_sc as plsc

N = 100000
E = 1600000
H = 64
HH = 32
NHALF = 50000
NPAD = 50048
DUMMY = 50040
NTILES = 16
ROWS_PER_TILE = NPAD // NTILES
ET = E // NTILES
CH = 2048
NCH = ET // CH
TAIL = ET - NCH * CH
SB = 128
NSB = CH // SB
TSB = TAIL // SB
TREM = TAIL - TSB * SB
NB = 4
ZROWS = 782

_mesh = plsc.VectorSubcoreMesh(core_axis_name="c", subcore_axis_name="s")


def _zero_2d(ref, rows, width):
    z = jnp.zeros((16,), jnp.float32)

    def body(i, _):
        for k in range(width // 16):
            ref[i, pl.ds(k * 16, 16)] = z
        return 0

    lax.fori_loop(0, rows, body, 0)


def _dstl_group(dstbuf, dstl, slot, off, lo, dummy_vec, valid_groups=8):
    for k in range(8):
        if k < valid_groups:
            d = dstbuf[pl.ds(off + k * 16, 16)]
            m = (d >= lo) & (d < lo + NHALF)
            dstl[slot, pl.ds(k * 16, 16)] = jnp.where(m, d - lo, dummy_vec)
        else:
            dstl[slot, pl.ds(k * 16, 16)] = dummy_vec


@functools.partial(
    pl.kernel,
    out_type=jax.ShapeDtypeStruct((N, 16), jnp.float32),
    mesh=_mesh,
    scratch_types=[
        pltpu.VMEM((CH,), jnp.int32),
        pltpu.VMEM((NB, SB), jnp.int32),
        pltpu.VMEM((SB, 16), jnp.float32),
        pltpu.VMEM((ZROWS, 16), jnp.float32),
        pltpu.VMEM_SHARED((NPAD, 16), jnp.float32),
        pltpu.SemaphoreType.DMA,
        pltpu.SemaphoreType.DMA,
        pltpu.SemaphoreType.DMA,
        pltpu.SemaphoreType.DMA,
    ],
)
def _sc_deg(ei, out, dstbuf, dstl, ones, zbuf, cnt, s0, s1, s2, s3):
    c = lax.axis_index("c")
    s = lax.axis_index("s")
    sems = [s0, s1, s2, s3]
    lo = c * NHALF
    dummy_vec = jnp.full((16,), DUMMY, jnp.int32)

    _zero_2d(zbuf, ZROWS, 16)
    one = jnp.ones((16,), jnp.float32)

    def ones_body(i, _):
        ones[i, pl.ds(0, 16)] = one
        return 0

    lax.fori_loop(0, SB, ones_body, 0)

    for j in range(4):
        pltpu.sync_copy(zbuf, cnt.at[pl.ds(s * ROWS_PER_TILE + j * ZROWS, ZROWS)])
    plsc.subcore_barrier()

    def do_sb(slot, off, valid_groups=8):
        _dstl_group(dstbuf, dstl, slot, off, lo, dummy_vec, valid_groups)
        return pltpu.async_copy(ones, cnt.at[dstl.at[slot]], sems[slot], add=True)

    def chunk_body(i, _):
        base = s * ET + i * CH
        pltpu.sync_copy(ei.at[1, pl.ds(base, CH)], dstbuf)
        descs = [None] * NB
        for b in range(NSB):
            slot = b % NB
            if descs[slot] is not None:
                descs[slot].wait()
            descs[slot] = do_sb(slot, b * SB)
        for slot in range(NB):
            descs[slot].wait()
        return 0

    lax.fori_loop(0, NCH, chunk_body, 0)

    base = s * ET + NCH * CH
    pltpu.sync_copy(ei.at[1, pl.ds(base, TAIL)], dstbuf.at[pl.ds(0, TAIL)])
    descs = [None] * NB
    for b in range(TSB):
        slot = b % NB
        if descs[slot] is not None:
            descs[slot].wait()
        descs[slot] = do_sb(slot, b * SB)
    slot = TSB % NB
    if descs[slot] is not None:
        descs[slot].wait()
    descs[slot] = do_sb(slot, TSB * SB, valid_groups=TREM // 16)
    for q in range(NB):
        if descs[q] is not None:
            descs[q].wait()
    plsc.subcore_barrier()

    row0 = s * ROWS_PER_TILE

    @pl.when(s < NTILES - 1)
    def _():
        pltpu.sync_copy(cnt.at[pl.ds(row0, ROWS_PER_TILE)],
                        out.at[pl.ds(lo + row0, ROWS_PER_TILE)])

    @pl.when(s == NTILES - 1)
    def _():
        last = NHALF - (NTILES - 1) * ROWS_PER_TILE
        pltpu.sync_copy(cnt.at[pl.ds((NTILES - 1) * ROWS_PER_TILE, last)],
                        out.at[pl.ds(lo + (NTILES - 1) * ROWS_PER_TILE, last)])


@functools.partial(
    pl.kernel,
    out_type=jax.ShapeDtypeStruct((2, N, HH), jnp.float32),
    mesh=_mesh,
    scratch_types=[
        pltpu.VMEM((CH,), jnp.int32),
        pltpu.VMEM((CH,), jnp.int32),
        pltpu.VMEM((NB, SB), jnp.int32),
        pltpu.VMEM((NB, SB, HH), jnp.float32),
        pltpu.VMEM((ZROWS, HH), jnp.float32),
        pltpu.VMEM_SHARED((NPAD, HH), jnp.float32),
        pltpu.SemaphoreType.DMA,
        pltpu.SemaphoreType.DMA,
        pltpu.SemaphoreType.DMA,
        pltpu.SemaphoreType.DMA,
    ],
)
def _sc_agg(y, ei, out, srcbuf, dstbuf, dstl, rows, zbuf, acc, s0, s1, s2, s3):
    c = lax.axis_index("c")
    s = lax.axis_index("s")
    sems = [s0, s1, s2, s3]
    dummy_vec = jnp.full((16,), DUMMY, jnp.int32)

    _zero_2d(zbuf, ZROWS, HH)
    zi = jnp.zeros((16,), jnp.int32)

    def zi_body(i, _):
        srcbuf[pl.ds(i * 16, 16)] = zi
        return 0

    lax.fori_loop(0, CH // 16, zi_body, 0)

    ytab = y.at[c]

    for p in range(2):
        lo = p * NHALF
        for j in range(4):
            pltpu.sync_copy(zbuf, acc.at[pl.ds(s * ROWS_PER_TILE + j * ZROWS, ZROWS)])
        plsc.subcore_barrier()

        def gather_sb(slot, off):
            return pltpu.async_copy(ytab.at[srcbuf.at[pl.ds(off, SB)]],
                                    rows.at[slot], sems[slot])

        def scatter_sb(slot, off, valid_groups=8):
            _dstl_group(dstbuf, dstl, slot, off, lo, dummy_vec, valid_groups)
            pltpu.sync_copy(rows.at[slot], acc.at[dstl.at[slot]], add=True)

        def chunk_body(i, _):
            base = s * ET + i * CH
            pltpu.sync_copy(ei.at[0, pl.ds(base, CH)], srcbuf)
            pltpu.sync_copy(ei.at[1, pl.ds(base, CH)], dstbuf)
            descs = [None] * NB
            for b in range(NB):
                descs[b] = gather_sb(b, b * SB)
            for b in range(NSB):
                slot = b % NB
                descs[slot].wait()
                scatter_sb(slot, b * SB)
                nxt = b + NB
                if nxt < NSB:
                    descs[slot] = gather_sb(slot, nxt * SB)
            return 0

        lax.fori_loop(0, NCH, chunk_body, 0)

        base = s * ET + NCH * CH
        pltpu.sync_copy(ei.at[0, pl.ds(base, TAIL)], srcbuf.at[pl.ds(0, TAIL)])
        pltpu.sync_copy(ei.at[1, pl.ds(base, TAIL)], dstbuf.at[pl.ds(0, TAIL)])
        descs = [None] * NB
        for b in range(NB):
            descs[b] = gather_sb(b, b * SB)
        for b in range(TSB + 1):
            slot = b % NB
            descs[slot].wait()
            descs[slot] = None
            if b < TSB:
                scatter_sb(slot, b * SB)
            else:
                scatter_sb(slot, b * SB, valid_groups=TREM // 16)
            nxt = b + NB
            if nxt < TSB + 1:
                descs[slot] = gather_sb(slot, nxt * SB)
        plsc.subcore_barrier()

        row0 = s * ROWS_PER_TILE

        @pl.when(s < NTILES - 1)
        def _():
            pltpu.sync_copy(acc.at[pl.ds(row0, ROWS_PER_TILE)],
                            out.at[c, pl.ds(lo + row0, ROWS_PER_TILE)])

        @pl.when(s == NTILES - 1)
        def _():
            last = NHALF - (NTILES - 1) * ROWS_PER_TILE
            pltpu.sync_copy(acc.at[pl.ds((NTILES - 1) * ROWS_PER_TILE, last)],
                            out.at[c, pl.ds(lo + (NTILES - 1) * ROWS_PER_TILE, last)])


RB = 2000
GRID = N // RB


def _ln_relu(conv, g, b):
    mu = jnp.mean(conv, axis=-1, keepdims=True)
    d = conv - mu
    var = jnp.mean(d * d, axis=-1, keepdims=True)
    return jnp.maximum(d * lax.rsqrt(var + 1e-5) * g + b, 0.0)


def _tc_b_body(x, pk, pkp, deg, wgx, wgpk, wgp, bg, w0x, w0pk, w0p,
               gate_o, dinv_o, y_o):
    dinv = lax.rsqrt(1.0 + deg[...][:, 0:1])
    xx, pp, qq = x[...], pk[...], pkp[...]
    g = jax.nn.sigmoid(
        jnp.dot(xx, wgx[...], preferred_element_type=jnp.float32)
        + jnp.dot(pp, wgpk[...], preferred_element_type=jnp.float32)
        + qq * wgp[...] + bg[...])
    xw = (jnp.dot(xx, w0x[...], preferred_element_type=jnp.float32)
          + jnp.dot(pp, w0pk[...], preferred_element_type=jnp.float32)
          + qq * w0p[...])
    y = xw * dinv
    gate_o[...] = g
    dinv_o[...] = dinv
    y_o[0] = y[:, :HH]
    y_o[1] = y[:, HH:]


def _tc_d_body(layer, agg, y, dinv, b, lng, lnb, aux, w, h_o, y_o):
    yfull = jnp.concatenate([y[0], y[1]], axis=-1)
    full = jnp.concatenate([agg[0], agg[1]], axis=-1) + yfull
    di = dinv[...]
    conv = full * di + b[...]
    t = _ln_relu(conv, lng[...], lnb[...])
    if layer == 0:
        h = t * aux[...]
    else:
        h = aux[...] + t
    h_o[...] = h
    xw = jnp.dot(h, w[...], preferred_element_type=jnp.float32)
    ynext = xw * di
    y_o[0] = ynext[:, :HH]
    y_o[1] = ynext[:, HH:]


def _tc_d2_body(agg, y, dinv, b, lng, lnb, hprev, wp1, bp1, wp2, bp2, pd_o):
    yfull = jnp.concatenate([y[0], y[1]], axis=-1)
    full = jnp.concatenate([agg[0], agg[1]], axis=-1) + yfull
    conv = full * dinv[...] + b[...]
    t = _ln_relu(conv, lng[...], lnb[...])
    h = hprev[...] + t
    z = jnp.maximum(jnp.dot(h, wp1[...], preferred_element_type=jnp.float32)
                    + bp1[...], 0.0)
    pd_o[...] = jnp.dot(z, wp2[...], preferred_element_type=jnp.float32) + bp2[...]


def _row_spec(width):
    return pl.BlockSpec((RB, width), lambda i: (i, 0))


def _y_spec():
    return pl.BlockSpec((2, RB, HH), lambda i: (0, i, 0))


def _full_spec(shape):
    return pl.BlockSpec(shape, lambda i: tuple(0 for _ in shape))


def _tc_b(x, pk, pkp, deg16, wgx, wgpk, wgp, bg, w0x, w0pk, w0p):
    return pl.pallas_call(
        _tc_b_body,
        grid=(GRID,),
        in_specs=[_row_spec(32), _row_spec(16), _row_spec(1), _row_spec(16),
                  _full_spec((32, H)), _full_spec((16, H)), _full_spec((1, H)),
                  _full_spec((1, H)), _full_spec((32, H)), _full_spec((16, H)),
                  _full_spec((1, H))],
        out_specs=[_row_spec(H), _row_spec(1), _y_spec()],
        out_shape=[jax.ShapeDtypeStruct((N, H), jnp.float32),
                   jax.ShapeDtypeStruct((N, 1), jnp.float32),
                   jax.ShapeDtypeStruct((2, N, HH), jnp.float32)],
    )(x, pk, pkp, deg16, wgx, wgpk, wgp, bg, w0x, w0pk, w0p)


def _tc_d(layer, agg, y, dinv, b, lng, lnb, aux, w):
    return pl.pallas_call(
        functools.partial(_tc_d_body, layer),
        grid=(GRID,),
        in_specs=[_y_spec(), _y_spec(), _row_spec(1), _full_spec((1, H)),
                  _full_spec((1, H)), _full_spec((1, H)), _row_spec(H),
                  _full_spec((H, H))],
        out_specs=[_row_spec(H), _y_spec()],
        out_shape=[jax.ShapeDtypeStruct((N, H), jnp.float32),
                   jax.ShapeDtypeStruct((2, N, HH), jnp.float32)],
    )(agg, y, dinv, b, lng, lnb, aux, w)


def _tc_d2(agg, y, dinv, b, lng, lnb, hprev, wp1, bp1, wp2, bp2):
    return pl.pallas_call(
        _tc_d2_body,
        grid=(GRID,),
        in_specs=[_y_spec(), _y_spec(), _row_spec(1), _full_spec((1, H)),
                  _full_spec((1, H)), _full_spec((1, H)), _row_spec(H),
                  _full_spec((H, HH)), _full_spec((1, HH)),
                  _full_spec((HH, 1)), _full_spec((1, 1))],
        out_specs=_row_spec(1),
        out_shape=jax.ShapeDtypeStruct((N, 1), jnp.float32),
    )(agg, y, dinv, b, lng, lnb, hprev, wp1, bp1, wp2, bp2)


def kernel(x, pk_embeddings, pk_predictions, edge_index, W_gate, b_gate,
           W0, b0, W1, b1, W2, b2, ln_g0, ln_b0, ln_g1, ln_b1, ln_g2, ln_b2,
           Wp1, bp1, Wp2, bp2):
    r = lambda v: v.reshape(1, -1)
    deg16 = _sc_deg(edge_index)
    gate, dinv, y0 = _tc_b(
        x, pk_embeddings, pk_predictions, deg16,
        W_gate[:32], W_gate[32:48], W_gate[48:49], r(b_gate),
        W0[:32], W0[32:48], W0[48:49])
    agg0 = _sc_agg(y0, edge_index)
    h0, y1 = _tc_d(0, agg0, y0, dinv, r(b0), r(ln_g0), r(ln_b0), gate, W1)
    agg1 = _sc_agg(y1, edge_index)
    h, y2 = _tc_d(1, agg1, y1, dinv, r(b1), r(ln_g1), r(ln_b1), h0, W2)
    agg2 = _sc_agg(y2, edge_index)
    pd = _tc_d2(agg2, y2, dinv, r(b2), r(ln_g2), r(ln_b2), h,
                Wp1, r(bp1), Wp2, r(bp2))
    return pd


# speedup vs baseline: 3.6987x; 3.6987x over previous
import functools

import jax
import jax.numpy as jnp
from jax import lax
from jax.experimental import pallas as pl
from jax.experimental.pallas import tpu as pltpu
from jax.experimental.pallas import tpu_sc as plsc

N = 100000
E = 1600000
H = 64
HH = 32
QW = 16
NHALF = 50000
NPAD = 50048
DUMMY = 50040
NTILES = 16
ROWS_PER_TILE = NPAD // NTILES
ET = E // NTILES
CH = 2048
NCH = ET // CH
TAIL = ET - NCH * CH
SB = 128
NSB = CH // SB
TSB = TAIL // SB
TREM = TAIL - TSB * SB
NB = 4

_mesh = plsc.VectorSubcoreMesh(core_axis_name="c", subcore_axis_name="s")


def _zero_2d(ref, rows, width):
    z = jnp.zeros((16,), jnp.float32)

    def body(i, _):
        for k in range(width // 16):
            ref[i, pl.ds(k * 16, 16)] = z
        return 0

    lax.fori_loop(0, rows, body, 0)


def _dstl_group(dstbuf, dstl, slot, off, lo, dummy_vec, valid_groups=8):
    for k in range(8):
        if k < valid_groups:
            d = dstbuf[pl.ds(off + k * 16, 16)]
            m = (d >= lo) & (d < lo + NHALF)
            dstl[slot, pl.ds(k * 16, 16)] = jnp.where(m, d - lo, dummy_vec)
        else:
            dstl[slot, pl.ds(k * 16, 16)] = dummy_vec


@functools.partial(
    pl.kernel,
    out_type=jax.ShapeDtypeStruct((N,), jnp.float32),
    mesh=_mesh,
    scratch_types=[
        pltpu.VMEM((CH,), jnp.int32),
        pltpu.VMEM((NB, SB), jnp.int32),
        pltpu.VMEM((SB,), jnp.float32),
        pltpu.VMEM((3136,), jnp.float32),
        pltpu.VMEM_SHARED((NPAD,), jnp.float32),
        pltpu.SemaphoreType.DMA,
        pltpu.SemaphoreType.DMA,
        pltpu.SemaphoreType.DMA,
        pltpu.SemaphoreType.DMA,
    ],
    compiler_params=pltpu.CompilerParams(use_tc_tiling_on_sc=False),
)
def _sc_deg(dsth, out, dstbuf, dstl, ones, zbuf, cnt, s0, s1, s2, s3):
    c = lax.axis_index("c")
    s = lax.axis_index("s")
    sems = [s0, s1, s2, s3]
    lo = c * NHALF
    dummy_vec = jnp.full((16,), DUMMY, jnp.int32)

    z = jnp.zeros((16,), jnp.float32)
    one = jnp.ones((16,), jnp.float32)

    def init_body(i, _):
        zbuf[pl.ds(i * 16, 16)] = z
        return 0

    lax.fori_loop(0, 3136 // 16, init_body, 0)

    def ones_body(i, _):
        ones[pl.ds(i * 16, 16)] = one
        return 0

    lax.fori_loop(0, SB // 16, ones_body, 0)

    pltpu.sync_copy(zbuf.at[pl.ds(0, ROWS_PER_TILE)],
                    cnt.at[pl.ds(s * ROWS_PER_TILE, ROWS_PER_TILE)])
    plsc.subcore_barrier()

    def do_sb(slot, off, valid_groups=8):
        _dstl_group(dstbuf, dstl, slot, off, lo, dummy_vec, valid_groups)
        return pltpu.async_copy(ones, cnt.at[dstl.at[slot]], sems[slot], add=True)

    def chunk_body(i, _):
        base = s * ET + i * CH
        pltpu.sync_copy(dsth.at[pl.ds(base, CH)], dstbuf)
        descs = [None] * NB
        for b in range(NSB):
            slot = b % NB
            if descs[slot] is not None:
                descs[slot].wait()
            descs[slot] = do_sb(slot, b * SB)
        for slot in range(NB):
            descs[slot].wait()
        return 0

    lax.fori_loop(0, NCH, chunk_body, 0)

    base = s * ET + NCH * CH
    pltpu.sync_copy(dsth.at[pl.ds(base, TAIL)], dstbuf.at[pl.ds(0, TAIL)])
    descs = [None] * NB
    for b in range(TSB):
        slot = b % NB
        if descs[slot] is not None:
            descs[slot].wait()
        descs[slot] = do_sb(slot, b * SB)
    slot = TSB % NB
    if descs[slot] is not None:
        descs[slot].wait()
    descs[slot] = do_sb(slot, TSB * SB, valid_groups=TREM // 16)
    for q in range(NB):
        if descs[q] is not None:
            descs[q].wait()
    plsc.subcore_barrier()

    row0 = s * ROWS_PER_TILE

    @pl.when(s < NTILES - 1)
    def _():
        pltpu.sync_copy(cnt.at[pl.ds(row0, ROWS_PER_TILE)],
                        zbuf.at[pl.ds(0, ROWS_PER_TILE)])
        pltpu.sync_copy(zbuf.at[pl.ds(0, ROWS_PER_TILE)],
                        out.at[pl.ds(lo + row0, ROWS_PER_TILE)])

    @pl.when(s == NTILES - 1)
    def _():
        last = NHALF - (NTILES - 1) * ROWS_PER_TILE
        pltpu.sync_copy(cnt.at[pl.ds((NTILES - 1) * ROWS_PER_TILE, last)],
                        zbuf.at[pl.ds(0, last)])
        pltpu.sync_copy(zbuf.at[pl.ds(0, last)],
                        out.at[pl.ds(lo + (NTILES - 1) * ROWS_PER_TILE, last)])


@functools.partial(
    pl.kernel,
    out_type=jax.ShapeDtypeStruct((4, N, QW), jnp.float32),
    mesh=_mesh,
    scratch_types=[
        pltpu.VMEM((CH,), jnp.int32),
        pltpu.VMEM((CH,), jnp.int32),
        pltpu.VMEM((NB, SB), jnp.int32),
        pltpu.VMEM((NB, SB, QW), jnp.float32),
        pltpu.VMEM((ROWS_PER_TILE, QW), jnp.float32),
        pltpu.VMEM_SHARED((NPAD, QW), jnp.float32),
        pltpu.SemaphoreType.DMA,
        pltpu.SemaphoreType.DMA,
        pltpu.SemaphoreType.DMA,
        pltpu.SemaphoreType.DMA,
    ],
    compiler_params=pltpu.CompilerParams(use_tc_tiling_on_sc=False),
)
def _sc_agg(y, srch, dsth, out, srcbuf, dstbuf, dstl, rows, stg, acc,
            s0, s1, s2, s3):
    c = lax.axis_index("c")
    s = lax.axis_index("s")
    sems = [s0, s1, s2, s3]
    dummy_vec = jnp.full((16,), DUMMY, jnp.int32)

    zi = jnp.zeros((16,), jnp.int32)

    def zi_body(i, _):
        srcbuf[pl.ds(i * 16, 16)] = zi
        return 0

    lax.fori_loop(0, CH // 16, zi_body, 0)

    def pass_body(pp, _):
        f = pp // 2
        p = pp % 2
        lo = p * NHALF
        plane = 2 * c + f
        ytab = y.at[plane]

        _zero_2d(stg, ROWS_PER_TILE, QW)
        pltpu.sync_copy(stg, acc.at[pl.ds(s * ROWS_PER_TILE, ROWS_PER_TILE)])
        plsc.subcore_barrier()

        def gather_sb(slot, off):
            return pltpu.async_copy(ytab.at[srcbuf.at[pl.ds(off, SB)]],
                                    rows.at[slot], sems[slot])

        def scatter_sb(slot, off, valid_groups=8):
            _dstl_group(dstbuf, dstl, slot, off, lo, dummy_vec, valid_groups)
            pltpu.sync_copy(rows.at[slot], acc.at[dstl.at[slot]], add=True)

        def chunk_body(i, _):
            base = s * ET + i * CH
            pltpu.sync_copy(srch.at[pl.ds(base, CH)], srcbuf)
            pltpu.sync_copy(dsth.at[pl.ds(base, CH)], dstbuf)
            descs = [None] * NB
            for b in range(NB):
                descs[b] = gather_sb(b, b * SB)
            for b in range(NSB):
                slot = b % NB
                descs[slot].wait()
                scatter_sb(slot, b * SB)
                nxt = b + NB
                if nxt < NSB:
                    descs[slot] = gather_sb(slot, nxt * SB)
            return 0

        lax.fori_loop(0, NCH, chunk_body, 0)

        base = s * ET + NCH * CH
        pltpu.sync_copy(srch.at[pl.ds(base, TAIL)], srcbuf.at[pl.ds(0, TAIL)])
        pltpu.sync_copy(dsth.at[pl.ds(base, TAIL)], dstbuf.at[pl.ds(0, TAIL)])
        descs = [None] * NB
        for b in range(NB):
            descs[b] = gather_sb(b, b * SB)
        for b in range(TSB + 1):
            slot = b % NB
            descs[slot].wait()
            descs[slot] = None
            if b < TSB:
                scatter_sb(slot, b * SB)
            else:
                scatter_sb(slot, b * SB, valid_groups=TREM // 16)
            nxt = b + NB
            if nxt < TSB + 1:
                descs[slot] = gather_sb(slot, nxt * SB)
        plsc.subcore_barrier()

        row0 = s * ROWS_PER_TILE

        @pl.when(s < NTILES - 1)
        def _():
            pltpu.sync_copy(acc.at[pl.ds(row0, ROWS_PER_TILE)], stg)
            pltpu.sync_copy(stg, out.at[plane, pl.ds(lo + row0, ROWS_PER_TILE)])

        @pl.when(s == NTILES - 1)
        def _():
            r0 = (NTILES - 1) * ROWS_PER_TILE
            last = NHALF - r0
            pltpu.sync_copy(acc.at[pl.ds(r0, last)], stg.at[pl.ds(0, last)])
            pltpu.sync_copy(stg.at[pl.ds(0, last)],
                            out.at[plane, pl.ds(lo + r0, last)])

        plsc.subcore_barrier()
        return 0

    lax.fori_loop(0, 4, pass_body, 0)


RB = 2000
GRID = N // RB


def _ln_relu(conv, g, b):
    mu = jnp.mean(conv, axis=-1, keepdims=True)
    d = conv - mu
    var = jnp.mean(d * d, axis=-1, keepdims=True)
    return jnp.maximum(d * lax.rsqrt(var + 1e-5) * g + b, 0.0)


def _tc_b_body(x, pk, pkp, deg, wgx, wgpk, wgp, bg, w0x, w0pk, w0p,
               gate_o, dinv_o, y_o):
    dinv = lax.rsqrt(1.0 + deg[...])
    xx, pp, qq = x[...], pk[...], pkp[...]
    g = jax.nn.sigmoid(
        jnp.dot(xx, wgx[...], preferred_element_type=jnp.float32)
        + jnp.dot(pp, wgpk[...], preferred_element_type=jnp.float32)
        + qq * wgp[...] + bg[...])
    xw = (jnp.dot(xx, w0x[...], preferred_element_type=jnp.float32)
          + jnp.dot(pp, w0pk[...], preferred_element_type=jnp.float32)
          + qq * w0p[...])
    y = xw * dinv
    gate_o[...] = g
    dinv_o[...] = dinv
    for j in range(4):
        y_o[j] = y[:, j * QW:(j + 1) * QW]


def _tc_d_body(layer, agg, y, dinv, b, lng, lnb, aux, w, h_o, y_o):
    yfull = jnp.concatenate([y[0], y[1], y[2], y[3]], axis=-1)
    full = jnp.concatenate([agg[0], agg[1], agg[2], agg[3]], axis=-1) + yfull
    di = dinv[...]
    conv = full * di + b[...]
    t = _ln_relu(conv, lng[...], lnb[...])
    if layer == 0:
        h = t * aux[...]
    else:
        h = aux[...] + t
    h_o[...] = h
    xw = jnp.dot(h, w[...], preferred_element_type=jnp.float32)
    ynext = xw * di
    for j in range(4):
        y_o[j] = ynext[:, j * QW:(j + 1) * QW]


def _tc_d2_body(agg, y, dinv, b, lng, lnb, hprev, wp1, bp1, wp2, bp2, pd_o):
    yfull = jnp.concatenate([y[0], y[1], y[2], y[3]], axis=-1)
    full = jnp.concatenate([agg[0], agg[1], agg[2], agg[3]], axis=-1) + yfull
    conv = full * dinv[...] + b[...]
    t = _ln_relu(conv, lng[...], lnb[...])
    h = hprev[...] + t
    z = jnp.maximum(jnp.dot(h, wp1[...], preferred_element_type=jnp.float32)
                    + bp1[...], 0.0)
    pd_o[...] = jnp.dot(z, wp2[...], preferred_element_type=jnp.float32) + bp2[...]


def _row_spec(width):
    return pl.BlockSpec((RB, width), lambda i: (i, 0))


def _y_spec():
    return pl.BlockSpec((4, RB, QW), lambda i: (0, i, 0))


def _full_spec(shape):
    return pl.BlockSpec(shape, lambda i: tuple(0 for _ in shape))


def _tc_b(x, pk, pkp, deg1, wgx, wgpk, wgp, bg, w0x, w0pk, w0p):
    return pl.pallas_call(
        _tc_b_body,
        grid=(GRID,),
        in_specs=[_row_spec(32), _row_spec(16), _row_spec(1), _row_spec(1),
                  _full_spec((32, H)), _full_spec((16, H)), _full_spec((1, H)),
                  _full_spec((1, H)), _full_spec((32, H)), _full_spec((16, H)),
                  _full_spec((1, H))],
        out_specs=[_row_spec(H), _row_spec(1), _y_spec()],
        out_shape=[jax.ShapeDtypeStruct((N, H), jnp.float32),
                   jax.ShapeDtypeStruct((N, 1), jnp.float32),
                   jax.ShapeDtypeStruct((4, N, QW), jnp.float32)],
    )(x, pk, pkp, deg1, wgx, wgpk, wgp, bg, w0x, w0pk, w0p)


def _tc_d(layer, agg, y, dinv, b, lng, lnb, aux, w):
    return pl.pallas_call(
        functools.partial(_tc_d_body, layer),
        grid=(GRID,),
        in_specs=[_y_spec(), _y_spec(), _row_spec(1), _full_spec((1, H)),
                  _full_spec((1, H)), _full_spec((1, H)), _row_spec(H),
                  _full_spec((H, H))],
        out_specs=[_row_spec(H), _y_spec()],
        out_shape=[jax.ShapeDtypeStruct((N, H), jnp.float32),
                   jax.ShapeDtypeStruct((4, N, QW), jnp.float32)],
    )(agg, y, dinv, b, lng, lnb, aux, w)


def _tc_d2(agg, y, dinv, b, lng, lnb, hprev, wp1, bp1, wp2, bp2):
    return pl.pallas_call(
        _tc_d2_body,
        grid=(GRID,),
        in_specs=[_y_spec(), _y_spec(), _row_spec(1), _full_spec((1, H)),
                  _full_spec((1, H)), _full_spec((1, H)), _row_spec(H),
                  _full_spec((H, HH)), _full_spec((1, HH)),
                  _full_spec((HH, 1)), _full_spec((1, 1))],
        out_specs=_row_spec(1),
        out_shape=jax.ShapeDtypeStruct((N, 1), jnp.float32),
    )(agg, y, dinv, b, lng, lnb, hprev, wp1, bp1, wp2, bp2)


def kernel(x, pk_embeddings, pk_predictions, edge_index, W_gate, b_gate,
           W0, b0, W1, b1, W2, b2, ln_g0, ln_b0, ln_g1, ln_b1, ln_g2, ln_b2,
           Wp1, bp1, Wp2, bp2):
    r = lambda v: v.reshape(1, -1)
    src_a = edge_index[0]
    dst_a = edge_index[1]
    deg = _sc_deg(dst_a).reshape(N, 1)
    gate, dinv, y0 = _tc_b(
        x, pk_embeddings, pk_predictions, deg,
        W_gate[:32], W_gate[32:48], W_gate[48:49], r(b_gate),
        W0[:32], W0[32:48], W0[48:49])
    agg0 = _sc_agg(y0, src_a, dst_a)
    h0, y1 = _tc_d(0, agg0, y0, dinv, r(b0), r(ln_g0), r(ln_b0), gate, W1)
    agg1 = _sc_agg(y1, src_a, dst_a)
    h, y2 = _tc_d(1, agg1, y1, dinv, r(b1), r(ln_g1), r(ln_b1), h0, W2)
    agg2 = _sc_agg(y2, src_a, dst_a)
    pd = _tc_d2(agg2, y2, dinv, r(b2), r(ln_g2), r(ln_b2), h,
                Wp1, r(bp1), Wp2, r(bp2))
    return pd


# speedup vs baseline: 3.7176x; 1.0051x over previous
import functools

import jax
import jax.numpy as jnp
from jax import lax
from jax.experimental import pallas as pl
from jax.experimental.pallas import tpu as pltpu
from jax.experimental.pallas import tpu_sc as plsc

N = 100000
E = 1600000
H = 64
HH = 32
QW = 16
NHALF = 50000
NPAD = 50048
DUMMY = 50040
NTILES = 16
ROWS_PER_TILE = NPAD // NTILES
ET = E // NTILES
CH = 2048
NCH = ET // CH
TAIL = ET - NCH * CH
SB = 128
NSB = CH // SB
TSB = TAIL // SB
TREM = TAIL - TSB * SB
NB = 4

_mesh = plsc.VectorSubcoreMesh(core_axis_name="c", subcore_axis_name="s")


def _zero_2d(ref, rows, width):
    z = jnp.zeros((16,), jnp.float32)

    def body(i, _):
        for k in range(width // 16):
            ref[i, pl.ds(k * 16, 16)] = z
        return 0

    lax.fori_loop(0, rows, body, 0)


def _dstl_group(dstbuf, dstl, slot, off, lo, dummy_vec, valid_groups=8):
    for k in range(8):
        if k < valid_groups:
            d = dstbuf[pl.ds(off + k * 16, 16)]
            m = (d >= lo) & (d < lo + NHALF)
            dstl[slot, pl.ds(k * 16, 16)] = jnp.where(m, d - lo, dummy_vec)
        else:
            dstl[slot, pl.ds(k * 16, 16)] = dummy_vec


@functools.partial(
    pl.kernel,
    out_type=jax.ShapeDtypeStruct((N,), jnp.float32),
    mesh=_mesh,
    scratch_types=[
        pltpu.VMEM((CH,), jnp.int32),
        pltpu.VMEM((NB, SB), jnp.int32),
        pltpu.VMEM((SB,), jnp.float32),
        pltpu.VMEM((3136,), jnp.float32),
        pltpu.VMEM_SHARED((NPAD,), jnp.float32),
        pltpu.SemaphoreType.DMA,
        pltpu.SemaphoreType.DMA,
        pltpu.SemaphoreType.DMA,
        pltpu.SemaphoreType.DMA,
    ],
    compiler_params=pltpu.CompilerParams(use_tc_tiling_on_sc=False),
)
def _sc_deg(dsth, out, dstbuf, dstl, ones, zbuf, cnt, s0, s1, s2, s3):
    c = lax.axis_index("c")
    s = lax.axis_index("s")
    sems = [s0, s1, s2, s3]
    lo = c * NHALF
    dummy_vec = jnp.full((16,), DUMMY, jnp.int32)

    z = jnp.zeros((16,), jnp.float32)
    one = jnp.ones((16,), jnp.float32)

    def init_body(i, _):
        zbuf[pl.ds(i * 16, 16)] = z
        return 0

    lax.fori_loop(0, 3136 // 16, init_body, 0)

    def ones_body(i, _):
        ones[pl.ds(i * 16, 16)] = one
        return 0

    lax.fori_loop(0, SB // 16, ones_body, 0)

    pltpu.sync_copy(zbuf.at[pl.ds(0, ROWS_PER_TILE)],
                    cnt.at[pl.ds(s * ROWS_PER_TILE, ROWS_PER_TILE)])
    plsc.subcore_barrier()

    def do_sb(slot, off, valid_groups=8):
        _dstl_group(dstbuf, dstl, slot, off, lo, dummy_vec, valid_groups)
        return pltpu.async_copy(ones, cnt.at[dstl.at[slot]], sems[slot], add=True)

    def chunk_body(i, _):
        base = s * ET + i * CH
        pltpu.sync_copy(dsth.at[pl.ds(base, CH)], dstbuf)
        descs = [None] * NB
        for b in range(NSB):
            slot = b % NB
            if descs[slot] is not None:
                descs[slot].wait()
            descs[slot] = do_sb(slot, b * SB)
        for slot in range(NB):
            descs[slot].wait()
        return 0

    lax.fori_loop(0, NCH, chunk_body, 0)

    base = s * ET + NCH * CH
    pltpu.sync_copy(dsth.at[pl.ds(base, TAIL)], dstbuf.at[pl.ds(0, TAIL)])
    descs = [None] * NB
    for b in range(TSB):
        slot = b % NB
        if descs[slot] is not None:
            descs[slot].wait()
        descs[slot] = do_sb(slot, b * SB)
    slot = TSB % NB
    if descs[slot] is not None:
        descs[slot].wait()
    descs[slot] = do_sb(slot, TSB * SB, valid_groups=TREM // 16)
    for q in range(NB):
        if descs[q] is not None:
            descs[q].wait()
    plsc.subcore_barrier()

    row0 = s * ROWS_PER_TILE

    @pl.when(s < NTILES - 1)
    def _():
        pltpu.sync_copy(cnt.at[pl.ds(row0, ROWS_PER_TILE)],
                        zbuf.at[pl.ds(0, ROWS_PER_TILE)])
        pltpu.sync_copy(zbuf.at[pl.ds(0, ROWS_PER_TILE)],
                        out.at[pl.ds(lo + row0, ROWS_PER_TILE)])

    @pl.when(s == NTILES - 1)
    def _():
        last = NHALF - (NTILES - 1) * ROWS_PER_TILE
        pltpu.sync_copy(cnt.at[pl.ds((NTILES - 1) * ROWS_PER_TILE, last)],
                        zbuf.at[pl.ds(0, last)])
        pltpu.sync_copy(zbuf.at[pl.ds(0, last)],
                        out.at[pl.ds(lo + (NTILES - 1) * ROWS_PER_TILE, last)])


ER = E // SB
CR = 8
NCHUNK = 96


def _tile_row_base(s):
    return s * 781 + jnp.minimum(s, 4)


@functools.partial(
    pl.kernel,
    out_type=jax.ShapeDtypeStruct((4, N, QW), jnp.float32),
    mesh=_mesh,
    scratch_types=[
        pltpu.VMEM((4, CR * SB), jnp.int32),
        pltpu.VMEM((4, CR * SB), jnp.int32),
        pltpu.VMEM((2, CR * SB), jnp.int32),
        pltpu.VMEM((2, CR * SB, QW), jnp.float32),
        pltpu.VMEM((1568, QW), jnp.float32),
        pltpu.VMEM_SHARED((NPAD, QW), jnp.float32),
        pltpu.SemaphoreType.DMA,
        pltpu.SemaphoreType.DMA,
        pltpu.SemaphoreType.DMA,
        pltpu.SemaphoreType.DMA,
        pltpu.SemaphoreType.DMA,
        pltpu.SemaphoreType.DMA,
        pltpu.SemaphoreType.DMA,
        pltpu.SemaphoreType.DMA,
    ],
    compiler_params=pltpu.CompilerParams(use_tc_tiling_on_sc=False),
)
def _sc_agg(y, srch, dsth, out, srcb, dstb, dstl, rows, stg, acc,
            g0, g1, t0, t1, i0, i1, i2, i3):
    c = lax.axis_index("c")
    s = lax.axis_index("s")
    gsem = [g0, g1]
    ssem = [t0, t1]
    isem = [i0, i1, i2, i3]
    dummy_vec = jnp.full((16,), DUMMY, jnp.int32)
    ebase = _tile_row_base(s) * SB

    def pass_body(pp, _):
        f = pp // 2
        p = pp % 2
        lo = p * NHALF
        plane = 2 * c + f
        ytab = y.at[plane]

        _zero_2d(stg, 1568, QW)
        pltpu.sync_copy(stg, acc.at[pl.ds(s * ROWS_PER_TILE, 1568)])
        pltpu.sync_copy(stg.at[pl.ds(0, 1560)],
                        acc.at[pl.ds(s * ROWS_PER_TILE + 1568, 1560)])
        plsc.subcore_barrier()

        def stage(chunk, slot):
            e0 = ebase + chunk * (CR * SB)
            pltpu.async_copy(srch.at[pl.ds(e0, CR * SB)], srcb.at[slot],
                             isem[slot])
            pltpu.async_copy(dsth.at[pl.ds(e0, CR * SB)], dstb.at[slot],
                             isem[slot])

        def wait_stage(slot):
            pltpu.make_async_copy(srch.at[pl.ds(0, CR * SB)], srcb.at[slot],
                                  isem[slot]).wait()
            pltpu.make_async_copy(dsth.at[pl.ds(0, CR * SB)], dstb.at[slot],
                                  isem[slot]).wait()

        def gather(s2, s4):
            pltpu.async_copy(ytab.at[srcb.at[s4]], rows.at[s2], gsem[s2])

        def wait_gather(s2, s4):
            pltpu.make_async_copy(ytab.at[srcb.at[s4]], rows.at[s2],
                                  gsem[s2]).wait()

        def compute_dstl(s2, s4):
            def rbody(rr, _):
                for k in range(SB // 16):
                    off = rr * SB + k * 16
                    d = dstb[s4, pl.ds(off, 16)]
                    m = (d >= lo) & (d < lo + NHALF)
                    dstl[s2, pl.ds(off, 16)] = jnp.where(
                        m, d - lo, dummy_vec)
                return 0
            lax.fori_loop(0, CR, rbody, 0)

        def scatter(s2):
            pltpu.async_copy(rows.at[s2], acc.at[dstl.at[s2]],
                             ssem[s2], add=True)

        def wait_scatter(s2):
            pltpu.make_async_copy(rows.at[s2], acc.at[dstl.at[s2]],
                                  ssem[s2]).wait()

        for q in range(4):
            stage(q, q)
        for q in (0, 1):
            wait_stage(q)
            gather(q % 2, q)

        for i in range(4):
            s4 = i % 4
            s2 = i % 2
            wait_gather(s2, s4)
            compute_dstl(s2, s4)
            scatter(s2)
            stage(i + 4, s4)
            j = i + 2
            js4 = j % 4
            js2 = j % 2
            wait_stage(js4)
            wait_scatter(js2)
            gather(js2, js4)

        def chunk_body(i4, _):
            for par in range(4):
                i = i4 * 4 + par
                s4 = par
                s2 = par % 2
                wait_gather(s2, s4)
                compute_dstl(s2, s4)
                scatter(s2)

                @pl.when(i + 4 < NCHUNK)
                def _():
                    stage(i + 4, s4)

                j = i + 2
                js4 = (par + 2) % 4
                js2 = js4 % 2

                @pl.when(j < NCHUNK)
                def _():
                    wait_stage(js4)
                    wait_scatter(js2)
                    gather(js2, js4)
            return 0

        lax.fori_loop(1, NCHUNK // 4, chunk_body, 0)

        for q in range(2):
            wait_scatter(q)

        def rem_rows(nrows, roff, slot):
            ne = nrows * SB
            e0 = ebase + roff * SB
            pltpu.async_copy(srch.at[pl.ds(e0, ne)],
                             srcb.at[slot].at[pl.ds(0, ne)], isem[slot])
            pltpu.async_copy(dsth.at[pl.ds(e0, ne)],
                             dstb.at[slot].at[pl.ds(0, ne)], isem[slot])
            pltpu.make_async_copy(srch.at[pl.ds(0, ne)],
                                  srcb.at[slot].at[pl.ds(0, ne)],
                                  isem[slot]).wait()
            pltpu.make_async_copy(dsth.at[pl.ds(0, ne)],
                                  dstb.at[slot].at[pl.ds(0, ne)],
                                  isem[slot]).wait()

            s2 = slot % 2

            def rbody(rr, _):
                for k in range(SB // 16):
                    off = rr * SB + k * 16
                    d = dstb[slot, pl.ds(off, 16)]
                    m = (d >= lo) & (d < lo + NHALF)
                    dstl[s2, pl.ds(off, 16)] = jnp.where(
                        m, d - lo, dummy_vec)
                return 0
            lax.fori_loop(0, nrows, rbody, 0)
            pltpu.async_copy(ytab.at[srcb.at[slot].at[pl.ds(0, ne)]],
                             rows.at[s2].at[pl.ds(0, ne)], gsem[s2])
            pltpu.make_async_copy(ytab.at[srcb.at[slot].at[pl.ds(0, ne)]],
                                  rows.at[s2].at[pl.ds(0, ne)],
                                  gsem[s2]).wait()
            pltpu.async_copy(rows.at[s2].at[pl.ds(0, ne)],
                             acc.at[dstl.at[s2].at[pl.ds(0, ne)]],
                             ssem[s2], add=True)
            pltpu.make_async_copy(rows.at[s2].at[pl.ds(0, ne)],
                                  acc.at[dstl.at[s2].at[pl.ds(0, ne)]],
                                  ssem[s2]).wait()

        rem_rows(8, NCHUNK * CR, 0)

        @pl.when(s < 4)
        def _():
            rem_rows(6, NCHUNK * CR + 8, 1)

        @pl.when(s >= 4)
        def _():
            rem_rows(5, NCHUNK * CR + 8, 1)

        plsc.subcore_barrier()

        row0 = s * ROWS_PER_TILE

        @pl.when(s < NTILES - 1)
        def _():
            pltpu.sync_copy(acc.at[pl.ds(row0, 1568)], stg)
            pltpu.sync_copy(stg, out.at[plane, pl.ds(lo + row0, 1568)])
            pltpu.sync_copy(acc.at[pl.ds(row0 + 1568, 1560)],
                            stg.at[pl.ds(0, 1560)])
            pltpu.sync_copy(stg.at[pl.ds(0, 1560)],
                            out.at[plane, pl.ds(lo + row0 + 1568, 1560)])

        @pl.when(s == NTILES - 1)
        def _():
            r0 = (NTILES - 1) * ROWS_PER_TILE
            pltpu.sync_copy(acc.at[pl.ds(r0, 1568)], stg)
            pltpu.sync_copy(stg, out.at[plane, pl.ds(lo + r0, 1568)])
            last = NHALF - r0 - 1568
            pltpu.sync_copy(acc.at[pl.ds(r0 + 1568, last)],
                            stg.at[pl.ds(0, last)])
            pltpu.sync_copy(stg.at[pl.ds(0, last)],
                            out.at[plane, pl.ds(lo + r0 + 1568, last)])

        plsc.subcore_barrier()
        return 0

    lax.fori_loop(0, 4, pass_body, 0)


RB = 2000
GRID = N // RB


def _ln_relu(conv, g, b):
    mu = jnp.mean(conv, axis=-1, keepdims=True)
    d = conv - mu
    var = jnp.mean(d * d, axis=-1, keepdims=True)
    return jnp.maximum(d * lax.rsqrt(var + 1e-5) * g + b, 0.0)


def _tc_b_body(x, pk, pkp, deg, wgx, wgpk, wgp, bg, w0x, w0pk, w0p,
               gate_o, dinv_o, y_o):
    dinv = lax.rsqrt(1.0 + deg[...])
    xx, pp, qq = x[...], pk[...], pkp[...]
    g = jax.nn.sigmoid(
        jnp.dot(xx, wgx[...], preferred_element_type=jnp.float32)
        + jnp.dot(pp, wgpk[...], preferred_element_type=jnp.float32)
        + qq * wgp[...] + bg[...])
    xw = (jnp.dot(xx, w0x[...], preferred_element_type=jnp.float32)
          + jnp.dot(pp, w0pk[...], preferred_element_type=jnp.float32)
          + qq * w0p[...])
    y = xw * dinv
    gate_o[...] = g
    dinv_o[...] = dinv
    for j in range(4):
        y_o[j] = y[:, j * QW:(j + 1) * QW]


def _tc_d_body(layer, agg, y, dinv, b, lng, lnb, aux, w, h_o, y_o):
    yfull = jnp.concatenate([y[0], y[1], y[2], y[3]], axis=-1)
    full = jnp.concatenate([agg[0], agg[1], agg[2], agg[3]], axis=-1) + yfull
    di = dinv[...]
    conv = full * di + b[...]
    t = _ln_relu(conv, lng[...], lnb[...])
    if layer == 0:
        h = t * aux[...]
    else:
        h = aux[...] + t
    h_o[...] = h
    xw = jnp.dot(h, w[...], preferred_element_type=jnp.float32)
    ynext = xw * di
    for j in range(4):
        y_o[j] = ynext[:, j * QW:(j + 1) * QW]


def _tc_d2_body(agg, y, dinv, b, lng, lnb, hprev, wp1, bp1, wp2, bp2, pd_o):
    yfull = jnp.concatenate([y[0], y[1], y[2], y[3]], axis=-1)
    full = jnp.concatenate([agg[0], agg[1], agg[2], agg[3]], axis=-1) + yfull
    conv = full * dinv[...] + b[...]
    t = _ln_relu(conv, lng[...], lnb[...])
    h = hprev[...] + t
    z = jnp.maximum(jnp.dot(h, wp1[...], preferred_element_type=jnp.float32)
                    + bp1[...], 0.0)
    pd_o[...] = jnp.dot(z, wp2[...], preferred_element_type=jnp.float32) + bp2[...]


def _row_spec(width):
    return pl.BlockSpec((RB, width), lambda i: (i, 0))


def _y_spec():
    return pl.BlockSpec((4, RB, QW), lambda i: (0, i, 0))


def _full_spec(shape):
    return pl.BlockSpec(shape, lambda i: tuple(0 for _ in shape))


def _tc_b(x, pk, pkp, deg1, wgx, wgpk, wgp, bg, w0x, w0pk, w0p):
    return pl.pallas_call(
        _tc_b_body,
        grid=(GRID,),
        in_specs=[_row_spec(32), _row_spec(16), _row_spec(1), _row_spec(1),
                  _full_spec((32, H)), _full_spec((16, H)), _full_spec((1, H)),
                  _full_spec((1, H)), _full_spec((32, H)), _full_spec((16, H)),
                  _full_spec((1, H))],
        out_specs=[_row_spec(H), _row_spec(1), _y_spec()],
        out_shape=[jax.ShapeDtypeStruct((N, H), jnp.float32),
                   jax.ShapeDtypeStruct((N, 1), jnp.float32),
                   jax.ShapeDtypeStruct((4, N, QW), jnp.float32)],
    )(x, pk, pkp, deg1, wgx, wgpk, wgp, bg, w0x, w0pk, w0p)


def _tc_d(layer, agg, y, dinv, b, lng, lnb, aux, w):
    return pl.pallas_call(
        functools.partial(_tc_d_body, layer),
        grid=(GRID,),
        in_specs=[_y_spec(), _y_spec(), _row_spec(1), _full_spec((1, H)),
                  _full_spec((1, H)), _full_spec((1, H)), _row_spec(H),
                  _full_spec((H, H))],
        out_specs=[_row_spec(H), _y_spec()],
        out_shape=[jax.ShapeDtypeStruct((N, H), jnp.float32),
                   jax.ShapeDtypeStruct((4, N, QW), jnp.float32)],
    )(agg, y, dinv, b, lng, lnb, aux, w)


def _tc_d2(agg, y, dinv, b, lng, lnb, hprev, wp1, bp1, wp2, bp2):
    return pl.pallas_call(
        _tc_d2_body,
        grid=(GRID,),
        in_specs=[_y_spec(), _y_spec(), _row_spec(1), _full_spec((1, H)),
                  _full_spec((1, H)), _full_spec((1, H)), _row_spec(H),
                  _full_spec((H, HH)), _full_spec((1, HH)),
                  _full_spec((HH, 1)), _full_spec((1, 1))],
        out_specs=_row_spec(1),
        out_shape=jax.ShapeDtypeStruct((N, 1), jnp.float32),
    )(agg, y, dinv, b, lng, lnb, hprev, wp1, bp1, wp2, bp2)


def kernel(x, pk_embeddings, pk_predictions, edge_index, W_gate, b_gate,
           W0, b0, W1, b1, W2, b2, ln_g0, ln_b0, ln_g1, ln_b1, ln_g2, ln_b2,
           Wp1, bp1, Wp2, bp2):
    r = lambda v: v.reshape(1, -1)
    src_a = edge_index[0]
    dst_a = edge_index[1]
    deg = _sc_deg(dst_a).reshape(N, 1)
    gate, dinv, y0 = _tc_b(
        x, pk_embeddings, pk_predictions, deg,
        W_gate[:32], W_gate[32:48], W_gate[48:49], r(b_gate),
        W0[:32], W0[32:48], W0[48:49])
    agg0 = _sc_agg(y0, src_a, dst_a)
    h0, y1 = _tc_d(0, agg0, y0, dinv, r(b0), r(ln_g0), r(ln_b0), gate, W1)
    agg1 = _sc_agg(y1, src_a, dst_a)
    h, y2 = _tc_d(1, agg1, y1, dinv, r(b1), r(ln_g1), r(ln_b1), h0, W2)
    agg2 = _sc_agg(y2, src_a, dst_a)
    pd = _tc_d2(agg2, y2, dinv, r(b2), r(ln_g2), r(ln_b2), h,
                Wp1, r(bp1), Wp2, r(bp2))
    return pd
